# baseline (speedup 1.0000x reference)
import dataclasses

import jax
import jax.numpy as jnp
from jax import lax
from jax.experimental import pallas as pl
from jax.experimental.pallas import tpu as pltpu
from jax.experimental.pallas import tpu_sc as plsc

H = 4
K = 128
NW = 32
RW = 16


def _f32(*shape):
    return jax.ShapeDtypeStruct(shape, jnp.float32)


_TC_PARAMS = pltpu.CompilerParams(vmem_limit_bytes=64 * 1024 * 1024)


def _sc_layer_call(src, dst, ts, td, zeros, *, oc, with_cnt):
    E = src.shape[0]
    N = ts.shape[0]
    rws = ts.shape[1]
    ho = H * oc
    A = zeros.shape[0]
    nch = E // K
    assert nch % (2 * NW) == 0
    npair = nch // (2 * NW)
    rz = A // 16
    nv = oc // 16
    aw = oc + 16 if with_cnt else oc

    mesh = plsc.VectorSubcoreMesh(
        core_axis_name="c", subcore_axis_name="s", num_cores=2, num_subcores=16
    )

    out_type = [_f32(2, A, aw)]
    scratch = [
        [pltpu.VMEM((K,), jnp.int32) for _ in range(2)],
        [pltpu.VMEM((K,), jnp.int32) for _ in range(2)],
        [pltpu.VMEM((K,), jnp.int32) for _ in range(2)],
        [pltpu.VMEM((K, rws), jnp.float32) for _ in range(2)],
        [pltpu.VMEM((K, RW), jnp.float32) for _ in range(2)],
        pltpu.VMEM((K, aw), jnp.float32),
        pltpu.VMEM((K * H,), jnp.float32),
        pltpu.VMEM_SHARED((A, aw), jnp.float32),
        pltpu.SemaphoreType.DMA,
    ]

    def body(src_hbm, dst_hbm, ts_hbm, td_hbm, z_hbm, pout,
             sidxs, didxs, dscs, xsrcs, xdsts, msg, q, acc, gsem):
        c = lax.axis_index("c")
        s = lax.axis_index("s")
        w = c * 16 + s

        pltpu.sync_copy(z_hbm.at[pl.ds(s * rz, rz)], acc.at[pl.ds(s * rz, rz)])

        pltpu.sync_copy(z_hbm.at[pl.ds(0, K)], msg)
        if with_cnt:
            one_hot = jnp.where(lax.iota(jnp.int32, 16) == 0, 1.0, 0.0)

            @pl.loop(0, K)
            def _(i):
                msg[i, pl.ds(oc, 16)] = one_hot

        plsc.subcore_barrier()

        @pl.loop(0, npair)
        def _(jp):
            ci0 = w + 2 * jp * NW

            gds = []
            for u in range(2):
                base = (ci0 + u * NW) * K
                pltpu.sync_copy(src_hbm.at[pl.ds(base, K)], sidxs[u])
                pltpu.sync_copy(dst_hbm.at[pl.ds(base, K)], didxs[u])
                gds.append(
                    pltpu.async_copy(ts_hbm.at[sidxs[u]], xsrcs[u], gsem))
                gds.append(
                    pltpu.async_copy(td_hbm.at[didxs[u]], xdsts[u], gsem))

                @pl.loop(0, K // 16)
                def _(k):
                    sv = sidxs[u][pl.ds(k * 16, 16)]
                    dv = didxs[u][pl.ds(k * 16, 16)]
                    dscs[u][pl.ds(k * 16, 16)] = jnp.where(sv == dv, N, dv)

            for d in gds:
                d.wait()
            for u in range(2):
                if True:
                    xsrc, xdst, dsc = xsrcs[u], xdsts[u], dscs[u]

                    @pl.loop(0, K // 16)
                    def _(g):
                        r = lax.iota(jnp.int32, 16) + g * 16
                        wgt = [
                            plsc.load_gather(xsrc, [r, jnp.full((16,), ho + h, jnp.int32)])
                            * plsc.load_gather(xdst, [r, jnp.full((16,), h, jnp.int32)])
                            for h in range(H)
                        ]
                        tot = (wgt[0] + wgt[1]) + (wgt[2] + wgt[3])
                        r_inv = 1.0 / tot
                        for h in range(H):
                            plsc.store_scatter(q, [r * H + h], wgt[h] * r_inv)

                    @pl.loop(0, K)
                    def _(e2):
                        accv = [None] * nv
                        for h in range(H):
                            qv = plsc.load_gather(
                                q, [jnp.full((16,), e2 * H + h, jnp.int32)])
                            for v in range(nv):
                                t = qv * xsrc[e2, pl.ds(h * oc + v * 16, 16)]
                                accv[v] = t if accv[v] is None else accv[v] + t
                        for v in range(nv):
                            msg[e2, pl.ds(v * 16, 16)] = accv[v]

                    pltpu.sync_copy(msg, acc.at[dsc], add=True)

        plsc.subcore_barrier()

        pltpu.sync_copy(acc.at[pl.ds(s * rz, rz)],
                        pout.at[c, pl.ds(s * rz, rz)])

    cp = pltpu.CompilerParams(use_tc_tiling_on_sc=False)
    if "needs_layout_passes" in pltpu.CompilerParams.__dataclass_fields__:
        cp = dataclasses.replace(cp, needs_layout_passes=False)
    call = pl.kernel(body, out_type=out_type, mesh=mesh, scratch_types=scratch,
                     compiler_params=cp)
    return call(src, dst, ts, td, zeros)


def _tables(h, w_ref, u_ref, c_ref, ws_ref, ts_ref, td_ref, sm_ref, rws):
    n = h.shape[0]
    ho = w_ref.shape[1]
    xw = jnp.dot(h, w_ref[...], preferred_element_type=jnp.float32)
    xu = jnp.dot(h, u_ref[...], preferred_element_type=jnp.float32)
    a = xu + c_ref[...]
    es = jnp.exp(a - jnp.max(a, axis=1, keepdims=True))
    ed = jnp.exp(jnp.min(xu, axis=1, keepdims=True) - xu)
    ts_ref[...] = jnp.concatenate(
        [xw, es, jnp.zeros((n, rws - ho - H), jnp.float32)], axis=1)
    td_ref[...] = jnp.concatenate(
        [ed, jnp.zeros((n, RW - H), jnp.float32)], axis=1)
    sm_ref[...] = jnp.dot(h, ws_ref[...], preferred_element_type=jnp.float32)


def _tc_pre(x, W, U, cvec, Wself):
    N = x.shape[0]
    ho = W.shape[1]
    oc = Wself.shape[1]
    rws = ho + 16

    def body(x_ref, w_ref, u_ref, c_ref, ws_ref, ts_ref, td_ref, sm_ref):
        _tables(x_ref[...], w_ref, u_ref, c_ref, ws_ref,
                ts_ref, td_ref, sm_ref, rws)

    return pl.pallas_call(
        body, out_shape=[_f32(N, rws), _f32(N, RW), _f32(N, oc)],
        compiler_params=_TC_PARAMS,
    )(x, W, U, cvec, Wself)


def _tc_mid(p, cntp, sm, bvec, W, U, cvec, Wself, *, ocp):
    N = sm.shape[0]
    ho = W.shape[1]
    oc = Wself.shape[1]
    rws = ho + 16

    def body(p_ref, cn_ref, sm_ref, b_ref, w_ref, u_ref, c_ref, ws_ref,
             ts_ref, td_ref, sm2_ref):
        cnt = (cn_ref[0, pl.ds(0, N), 16:17] + cn_ref[1, pl.ds(0, N), 16:17]) + 1.0
        invc = 1.0 / jnp.maximum(cnt, 1.0)
        ssum = (p_ref[0, pl.ds(0, N), pl.ds(0, ocp)]
                + p_ref[1, pl.ds(0, N), pl.ds(0, ocp)] + sm_ref[...])
        h = jnp.maximum(ssum * invc + b_ref[...], 0.0)
        _tables(h, w_ref, u_ref, c_ref, ws_ref, ts_ref, td_ref, sm2_ref, rws)

    return pl.pallas_call(
        body, out_shape=[_f32(N, rws), _f32(N, RW), _f32(N, oc)],
        compiler_params=_TC_PARAMS,
    )(p, cntp, sm, bvec, W, U, cvec, Wself)


def _tc_head(p, cntp, sm, bvec, gam, bet, lw1, lb1, lw2, lb2, lw3, lb3,
             lw4, lb4, lwo, lbo):
    N = sm.shape[0]
    ocp = sm.shape[1]

    def body(p_ref, cn_ref, sm_ref, b_ref, g_ref, be_ref, w1_ref, b1_ref,
             w2_ref, b2_ref, w3_ref, b3_ref, w4_ref, b4_ref, wo_ref, bo_ref,
             o_ref):
        cnt = (cn_ref[0, pl.ds(0, N), 16:17] + cn_ref[1, pl.ds(0, N), 16:17]) + 1.0
        invc = 1.0 / jnp.maximum(cnt, 1.0)
        ssum = (p_ref[0, pl.ds(0, N), pl.ds(0, ocp)]
                + p_ref[1, pl.ds(0, N), pl.ds(0, ocp)] + sm_ref[...])
        h = jnp.maximum(ssum * invc + b_ref[...], 0.0)
        mean = jnp.mean(h, axis=0, keepdims=True)
        var = jnp.mean((h - mean) ** 2, axis=0, keepdims=True)
        h = (h - mean) / jnp.sqrt(var + 1e-5) * g_ref[...] + be_ref[...]
        h = jnp.maximum(jnp.dot(h, w1_ref[...], preferred_element_type=jnp.float32) + b1_ref[...], 0.0)
        h = jnp.maximum(jnp.dot(h, w2_ref[...], preferred_element_type=jnp.float32) + b2_ref[...], 0.0)
        h = jnp.maximum(jnp.dot(h, w3_ref[...], preferred_element_type=jnp.float32) + b3_ref[...], 0.0)
        h = jnp.maximum(jnp.dot(h, w4_ref[...], preferred_element_type=jnp.float32) + b4_ref[...], 0.0)
        z = jnp.dot(h, wo_ref[...], preferred_element_type=jnp.float32) + bo_ref[...]
        o_ref[...] = 1.0 / (1.0 + jnp.exp(-z))

    return pl.pallas_call(body, out_shape=_f32(N, 1),
                          compiler_params=_TC_PARAMS)(
        p, cntp, sm, bvec, gam, bet, lw1, lb1, lw2, lb2, lw3, lb3,
        lw4, lb4, lwo, lbo)


def _wself(W, c):
    f, ho = W.shape
    qs = jax.nn.softmax(c)
    return jnp.einsum("fho,h->fo", W.reshape(f, H, ho // H), qs)


def kernel(x, edge_index, W1, U1, c1, b1, W2, U2, c2, b2, W3, U3, c3, b3,
           gamma, beta, lw1, lb1, lw2, lb2, lw3, lb3, lw4, lb4, lwo, lbo):
    N = x.shape[0]
    A = N + 112
    E = edge_index.shape[1]
    epad = -(-E // (K * 2 * NW)) * (K * 2 * NW)
    zi = jnp.zeros((epad - E,), edge_index.dtype)
    src = jnp.concatenate([edge_index[0], zi])
    dst = jnp.concatenate([edge_index[1], zi])

    ws1, ws2, ws3 = _wself(W1, c1), _wself(W2, c2), _wself(W3, c3)
    z1 = jnp.zeros((A, 32), jnp.float32)
    z2 = jnp.zeros((A, 32), jnp.float32)
    z3 = jnp.zeros((A, 64), jnp.float32)

    ts1, td1, sm1 = _tc_pre(x, W1, U1, c1.reshape(1, H), ws1)
    (p1,) = _sc_layer_call(src, dst, ts1, td1, z1, oc=16, with_cnt=True)
    ts2, td2, sm2 = _tc_mid(p1, p1, sm1, b1.reshape(1, 16),
                            W2, U2, c2.reshape(1, H), ws2, ocp=16)
    (p2,) = _sc_layer_call(src, dst, ts2, td2, z2, oc=32, with_cnt=False)
    ts3, td3, sm3 = _tc_mid(p2, p1, sm2, b2.reshape(1, 32),
                            W3, U3, c3.reshape(1, H), ws3, ocp=32)
    (p3,) = _sc_layer_call(src, dst, ts3, td3, z3, oc=64, with_cnt=False)
    return _tc_head(p3, p1, sm3, b3.reshape(1, 64),
                    gamma.reshape(1, 64), beta.reshape(1, 64),
                    lw1, lb1.reshape(1, 32), lw2, lb2.reshape(1, 16),
                    lw3, lb3.reshape(1, 8), lw4, lb4.reshape(1, 4),
                    lwo, lbo.reshape(1, 1))


# speedup vs baseline: 1.4623x; 1.4623x over previous
import dataclasses

import jax
import jax.numpy as jnp
from jax import lax
from jax.experimental import pallas as pl
from jax.experimental.pallas import tpu as pltpu
from jax.experimental.pallas import tpu_sc as plsc

H = 4
K = 128
NW = 32
RW = 16


def _f32(*shape):
    return jax.ShapeDtypeStruct(shape, jnp.float32)


_TC_PARAMS = pltpu.CompilerParams(vmem_limit_bytes=64 * 1024 * 1024)


def _sc_layer_call(src, dst, ts, td, zeros, *, oc, with_cnt):
    E = src.shape[0]
    N = ts.shape[0]
    rws = ts.shape[1]
    ho = H * oc
    A = zeros.shape[0]
    nch = E // K
    nj = (nch + NW - 1) // NW
    rz = A // 16
    nv = oc // 16
    aw = oc + 16 if with_cnt else oc

    mesh = plsc.VectorSubcoreMesh(
        core_axis_name="c", subcore_axis_name="s", num_cores=2, num_subcores=16
    )

    out_type = [_f32(2, A, aw)]
    scratch = [
        pltpu.VMEM((K,), jnp.int32),
        pltpu.VMEM((K,), jnp.int32),
        pltpu.VMEM((K,), jnp.int32),
        pltpu.VMEM((K, rws), jnp.float32),
        pltpu.VMEM((K, RW), jnp.float32),
        pltpu.VMEM((K, aw), jnp.float32),
        pltpu.VMEM((K * H,), jnp.float32),
        pltpu.VMEM_SHARED((A, aw), jnp.float32),
        pltpu.SemaphoreType.DMA,
        pltpu.SemaphoreType.DMA,
    ]

    def body(src_hbm, dst_hbm, ts_hbm, td_hbm, z_hbm, pout,
             sidx, didx, dsc, xsrc, xdst, msg, q, acc, s1, s2):
        c = lax.axis_index("c")
        s = lax.axis_index("s")
        w = c * 16 + s

        pltpu.sync_copy(z_hbm.at[pl.ds(s * rz, rz)], acc.at[pl.ds(s * rz, rz)])

        pltpu.sync_copy(z_hbm.at[pl.ds(0, K)], msg)
        if with_cnt:
            one_hot = jnp.where(lax.iota(jnp.int32, 16) == 0, 1.0, 0.0)

            @pl.loop(0, K)
            def _(i):
                msg[i, pl.ds(oc, 16)] = one_hot

        plsc.subcore_barrier()

        @pl.loop(0, nj)
        def _(j):
            ci = w + j * NW

            @pl.when(ci < nch)
            def _():
                base = ci * K
                pltpu.sync_copy(src_hbm.at[pl.ds(base, K)], sidx)
                pltpu.sync_copy(dst_hbm.at[pl.ds(base, K)], didx)
                d1 = pltpu.async_copy(ts_hbm.at[sidx], xsrc, s1)
                d2 = pltpu.async_copy(td_hbm.at[didx], xdst, s2)

                @pl.loop(0, K // 16)
                def _(k):
                    sv = sidx[pl.ds(k * 16, 16)]
                    dv = didx[pl.ds(k * 16, 16)]
                    dsc[pl.ds(k * 16, 16)] = jnp.where(sv == dv, N, dv)

                d1.wait()
                d2.wait()

                @pl.loop(0, K // 16)
                def _(g):
                    r = lax.iota(jnp.int32, 16) + g * 16
                    wgt = [
                        plsc.load_gather(xsrc, [r, jnp.full((16,), ho + h, jnp.int32)])
                        * plsc.load_gather(xdst, [r, jnp.full((16,), h, jnp.int32)])
                        for h in range(H)
                    ]
                    tot = (wgt[0] + wgt[1]) + (wgt[2] + wgt[3])
                    r_inv = 1.0 / tot
                    for h in range(H):
                        plsc.store_scatter(q, [r * H + h], wgt[h] * r_inv)

                @pl.loop(0, K)
                def _(e2):
                    accv = [None] * nv
                    for h in range(H):
                        qv = plsc.load_gather(
                            q, [jnp.full((16,), e2 * H + h, jnp.int32)])
                        for v in range(nv):
                            t = qv * xsrc[e2, pl.ds(h * oc + v * 16, 16)]
                            accv[v] = t if accv[v] is None else accv[v] + t
                    for v in range(nv):
                        msg[e2, pl.ds(v * 16, 16)] = accv[v]

                pltpu.sync_copy(msg, acc.at[dsc], add=True)

        plsc.subcore_barrier()

        pltpu.sync_copy(acc.at[pl.ds(s * rz, rz)],
                        pout.at[c, pl.ds(s * rz, rz)])

    cp = pltpu.CompilerParams(use_tc_tiling_on_sc=False)
    if "needs_layout_passes" in pltpu.CompilerParams.__dataclass_fields__:
        cp = dataclasses.replace(cp, needs_layout_passes=False)
    call = pl.kernel(body, out_type=out_type, mesh=mesh, scratch_types=scratch,
                     compiler_params=cp)
    return call(src, dst, ts, td, zeros)


def _tables(h, w_ref, u_ref, c_ref, ws_ref, ts_ref, td_ref, sm_ref, rws):
    n = h.shape[0]
    ho = w_ref.shape[1]
    xw = jnp.dot(h, w_ref[...], preferred_element_type=jnp.float32)
    xu = jnp.dot(h, u_ref[...], preferred_element_type=jnp.float32)
    a = xu + c_ref[...]
    es = jnp.exp(a - jnp.max(a, axis=1, keepdims=True))
    ed = jnp.exp(jnp.min(xu, axis=1, keepdims=True) - xu)
    ts_ref[...] = jnp.concatenate(
        [xw, es, jnp.zeros((n, rws - ho - H), jnp.float32)], axis=1)
    td_ref[...] = jnp.concatenate(
        [ed, jnp.zeros((n, RW - H), jnp.float32)], axis=1)
    sm_ref[...] = jnp.dot(h, ws_ref[...], preferred_element_type=jnp.float32)


def _tc_pre(x, W, U, cvec, Wself):
    N = x.shape[0]
    ho = W.shape[1]
    oc = Wself.shape[1]
    rws = ho + 16

    def body(x_ref, w_ref, u_ref, c_ref, ws_ref, ts_ref, td_ref, sm_ref):
        _tables(x_ref[...], w_ref, u_ref, c_ref, ws_ref,
                ts_ref, td_ref, sm_ref, rws)

    return pl.pallas_call(
        body, out_shape=[_f32(N, rws), _f32(N, RW), _f32(N, oc)],
        compiler_params=_TC_PARAMS,
    )(x, W, U, cvec, Wself)


def _tc_mid(p, cntp, sm, bvec, W, U, cvec, Wself, *, ocp):
    N = sm.shape[0]
    ho = W.shape[1]
    oc = Wself.shape[1]
    rws = ho + 16

    def body(p_ref, cn_ref, sm_ref, b_ref, w_ref, u_ref, c_ref, ws_ref,
             ts_ref, td_ref, sm2_ref):
        cnt = (cn_ref[0, pl.ds(0, N), 16:17] + cn_ref[1, pl.ds(0, N), 16:17]) + 1.0
        invc = 1.0 / jnp.maximum(cnt, 1.0)
        ssum = (p_ref[0, pl.ds(0, N), pl.ds(0, ocp)]
                + p_ref[1, pl.ds(0, N), pl.ds(0, ocp)] + sm_ref[...])
        h = jnp.maximum(ssum * invc + b_ref[...], 0.0)
        _tables(h, w_ref, u_ref, c_ref, ws_ref, ts_ref, td_ref, sm2_ref, rws)

    return pl.pallas_call(
        body, out_shape=[_f32(N, rws), _f32(N, RW), _f32(N, oc)],
        compiler_params=_TC_PARAMS,
    )(p, cntp, sm, bvec, W, U, cvec, Wself)


def _tc_head(p, cntp, sm, bvec, gam, bet, lw1, lb1, lw2, lb2, lw3, lb3,
             lw4, lb4, lwo, lbo):
    N = sm.shape[0]
    ocp = sm.shape[1]

    def body(p_ref, cn_ref, sm_ref, b_ref, g_ref, be_ref, w1_ref, b1_ref,
             w2_ref, b2_ref, w3_ref, b3_ref, w4_ref, b4_ref, wo_ref, bo_ref,
             o_ref):
        cnt = (cn_ref[0, pl.ds(0, N), 16:17] + cn_ref[1, pl.ds(0, N), 16:17]) + 1.0
        invc = 1.0 / jnp.maximum(cnt, 1.0)
        ssum = (p_ref[0, pl.ds(0, N), pl.ds(0, ocp)]
                + p_ref[1, pl.ds(0, N), pl.ds(0, ocp)] + sm_ref[...])
        h = jnp.maximum(ssum * invc + b_ref[...], 0.0)
        mean = jnp.mean(h, axis=0, keepdims=True)
        var = jnp.mean((h - mean) ** 2, axis=0, keepdims=True)
        h = (h - mean) / jnp.sqrt(var + 1e-5) * g_ref[...] + be_ref[...]
        h = jnp.maximum(jnp.dot(h, w1_ref[...], preferred_element_type=jnp.float32) + b1_ref[...], 0.0)
        h = jnp.maximum(jnp.dot(h, w2_ref[...], preferred_element_type=jnp.float32) + b2_ref[...], 0.0)
        h = jnp.maximum(jnp.dot(h, w3_ref[...], preferred_element_type=jnp.float32) + b3_ref[...], 0.0)
        h = jnp.maximum(jnp.dot(h, w4_ref[...], preferred_element_type=jnp.float32) + b4_ref[...], 0.0)
        z = jnp.dot(h, wo_ref[...], preferred_element_type=jnp.float32) + bo_ref[...]
        o_ref[...] = 1.0 / (1.0 + jnp.exp(-z))

    return pl.pallas_call(body, out_shape=_f32(N, 1),
                          compiler_params=_TC_PARAMS)(
        p, cntp, sm, bvec, gam, bet, lw1, lb1, lw2, lb2, lw3, lb3,
        lw4, lb4, lwo, lbo)


def _wself(W, c):
    f, ho = W.shape
    qs = jax.nn.softmax(c)
    return jnp.einsum("fho,h->fo", W.reshape(f, H, ho // H), qs)


def kernel(x, edge_index, W1, U1, c1, b1, W2, U2, c2, b2, W3, U3, c3, b3,
           gamma, beta, lw1, lb1, lw2, lb2, lw3, lb3, lw4, lb4, lwo, lbo):
    N = x.shape[0]
    A = N + 112
    src = edge_index[0]
    dst = edge_index[1]

    ws1, ws2, ws3 = _wself(W1, c1), _wself(W2, c2), _wself(W3, c3)
    z1 = jnp.zeros((A, 32), jnp.float32)
    z2 = jnp.zeros((A, 32), jnp.float32)
    z3 = jnp.zeros((A, 64), jnp.float32)

    ts1, td1, sm1 = _tc_pre(x, W1, U1, c1.reshape(1, H), ws1)
    (p1,) = _sc_layer_call(src, dst, ts1, td1, z1, oc=16, with_cnt=True)
    ts2, td2, sm2 = _tc_mid(p1, p1, sm1, b1.reshape(1, 16),
                            W2, U2, c2.reshape(1, H), ws2, ocp=16)
    (p2,) = _sc_layer_call(src, dst, ts2, td2, z2, oc=32, with_cnt=False)
    ts3, td3, sm3 = _tc_mid(p2, p1, sm2, b2.reshape(1, 32),
                            W3, U3, c3.reshape(1, H), ws3, ocp=32)
    (p3,) = _sc_layer_call(src, dst, ts3, td3, z3, oc=64, with_cnt=False)
    return _tc_head(p3, p1, sm3, b3.reshape(1, 64),
                    gamma.reshape(1, 64), beta.reshape(1, 64),
                    lw1, lb1.reshape(1, 32), lw2, lb2.reshape(1, 16),
                    lw3, lb3.reshape(1, 8), lw4, lb4.reshape(1, 4),
                    lwo, lbo.reshape(1, 1))
